# baseline (speedup 1.0000x reference)
import functools

import jax
import jax.numpy as jnp
from jax.experimental import pallas as pl
from jax.experimental.pallas import tpu as pltpu

_N = 10000
_E = 320000
_D = 128
_H1 = 4
_F1 = 1024
_C2 = 256

_NB = 400
_KA = 16
_KB = 8


def _dense1_body(x_ref, w1_ref, wl1_ref, asrc_ref, adst_ref, bfold_ref,
                 h1_ref, as_ref, ad_ref, xl1_ref):
    x = x_ref[...]
    h = jnp.dot(x, w1_ref[...], preferred_element_type=jnp.float32)
    h1_ref[...] = h
    xl1_ref[...] = (
        jnp.dot(x, wl1_ref[...], preferred_element_type=jnp.float32)
        + bfold_ref[...])
    ps = h * asrc_ref[...]
    pd = h * adst_ref[...]
    as_cols = [jnp.sum(ps[:, i * 256:(i + 1) * 256], axis=1, keepdims=True)
               for i in range(_H1)]
    ad_cols = [jnp.sum(pd[:, i * 256:(i + 1) * 256], axis=1, keepdims=True)
               for i in range(_H1)]
    as_ref[...] = jnp.concatenate(as_cols, axis=1)
    ad_ref[...] = jnp.concatenate(ad_cols, axis=1)


def _dense2_body(g1a_ref, g1b_ref, g1c_ref, g1d_ref, xl1_ref, w2_ref,
                 wl2_ref, asrc2_ref, adst2_ref, bfold2_ref,
                 h2_ref, as2_ref, ad2_ref, xl2_ref):
    x1 = (g1a_ref[...] + g1b_ref[...] + g1c_ref[...] + g1d_ref[...]
          + xl1_ref[...])
    x1 = jnp.where(x1 > 0, x1, jnp.exp(jnp.minimum(x1, 0.0)) - 1.0)
    h2 = jnp.dot(x1, w2_ref[...], preferred_element_type=jnp.float32)
    h2_ref[...] = h2
    xl2_ref[...] = (
        jnp.dot(x1, wl2_ref[...], preferred_element_type=jnp.float32)
        + bfold2_ref[...])
    as2_ref[...] = jnp.sum(h2 * asrc2_ref[...], axis=1, keepdims=True)
    ad2_ref[...] = jnp.sum(h2 * adst2_ref[...], axis=1, keepdims=True)


def _edge_weight_body(src_ref, dst_ref, as_ref, ad_ref, ee_ref, denom_ref, *,
                      k):
    i = pl.program_id(0)

    @pl.when(i == 0)
    def _():
        denom_ref[...] = jnp.zeros_like(denom_ref)

    base = i * k
    for j in range(k):
        s = src_ref[base + j]
        d = dst_ref[base + j]
        e = as_ref[pl.ds(s, 1)] + ad_ref[pl.ds(d, 1)]
        e = jnp.where(e >= 0, e, 0.2 * e)
        ee = jnp.exp(e)
        ee_ref[pl.ds(j, 1)] = ee
        denom_ref[pl.ds(d, 1)] = denom_ref[pl.ds(d, 1)] + ee


def _edge_agg_body(src_ref, *args, k, nheads, hs):
    hrefs = args[:k]
    dst_ref, ee_ref, denom_ref, alpha_ref, out_ref = args[k:]
    i = pl.program_id(0)

    @pl.when(i == 0)
    def _():
        out_ref[...] = jnp.zeros_like(out_ref)

    base = i * k
    for j in range(k):
        d = dst_ref[base + j]
        ee = ee_ref[pl.ds(j, 1)]
        den = denom_ref[pl.ds(d, 1)]
        alpha = ee / (den + 1e-16)
        alpha_ref[pl.ds(j, 1)] = alpha
        for h in range(nheads):
            msg = (hrefs[j][pl.ds(0, 1), pl.ds(h * hs, hs), :]
                   * alpha[:, :, h:h + 1])
            out_ref[pl.ds(d, 1), pl.ds(h * hs, hs), :] = (
                out_ref[pl.ds(d, 1), pl.ds(h * hs, hs), :] + msg)


def _final_body(g2a_ref, g2b_ref, g2c_ref, g2d_ref, xl2_ref, x2_ref):
    x2_ref[...] = (g2a_ref[...] + g2b_ref[...] + g2c_ref[...]
                   + g2d_ref[...] + xl2_ref[...])


def _add4_body(a_ref, b_ref, c_ref, d_ref, o_ref):
    o_ref[...] = a_ref[...] + b_ref[...] + c_ref[...] + d_ref[...]


def _edge_weight_pass(src, dst, as_, ad_, nheads):
    half = _E // 4
    ees, denoms = [], []
    for c in range(4):
        lo = c * half
        ee_c, denom_c = pl.pallas_call(
            functools.partial(_edge_weight_body, k=_KA),
            grid=(half // _KA,),
            in_specs=[
                pl.BlockSpec((half,), lambda i: (0,),
                             memory_space=pltpu.SMEM),
                pl.BlockSpec((half,), lambda i: (0,),
                             memory_space=pltpu.SMEM),
                pl.BlockSpec((_N, 1, nheads), lambda i: (0, 0, 0)),
                pl.BlockSpec((_N, 1, nheads), lambda i: (0, 0, 0)),
            ],
            out_specs=[
                pl.BlockSpec((_KA, 1, nheads), lambda i: (i, 0, 0)),
                pl.BlockSpec((_N, 1, nheads), lambda i: (0, 0, 0)),
            ],
            out_shape=[
                jax.ShapeDtypeStruct((half, 1, nheads), jnp.float32),
                jax.ShapeDtypeStruct((_N, 1, nheads), jnp.float32),
            ],
        )(jax.lax.slice(src, (lo,), (lo + half,)),
          jax.lax.slice(dst, (lo,), (lo + half,)), as_, ad_)
        ees.append(ee_c)
        denoms.append(denom_c)
    denom = pl.pallas_call(
        _add4_body,
        out_shape=jax.ShapeDtypeStruct((_N, 1, nheads), jnp.float32),
    )(*denoms)
    return jnp.concatenate(ees, axis=0), denom


def _edge_agg_chunk(src_c, dst_c, h3, ee_c, denom, nheads, nsub, nedges):
    hs = nsub // nheads
    h_specs = [
        pl.BlockSpec((1, nsub, 128),
                     (lambda i, s, jj=jj: (s[i * _KB + jj], 0, 0)))
        for jj in range(_KB)
    ]
    in_specs = h_specs + [
        pl.BlockSpec((nedges,), lambda i, s: (0,),
                     memory_space=pltpu.SMEM),
        pl.BlockSpec((_KB, 1, nheads), lambda i, s: (i, 0, 0)),
        pl.BlockSpec((_N, 1, nheads), lambda i, s: (0, 0, 0)),
    ]
    operands = [h3] * _KB + [dst_c, ee_c, denom]
    grid_spec = pltpu.PrefetchScalarGridSpec(
        num_scalar_prefetch=1,
        grid=(nedges // _KB,),
        in_specs=in_specs,
        out_specs=[
            pl.BlockSpec((_KB, 1, nheads), lambda i, s: (i, 0, 0)),
            pl.BlockSpec((_N, nsub, 128), lambda i, s: (0, 0, 0)),
        ],
    )
    return pl.pallas_call(
        functools.partial(_edge_agg_body, k=_KB, nheads=nheads, hs=hs),
        grid_spec=grid_spec,
        out_shape=[
            jax.ShapeDtypeStruct((nedges, 1, nheads), jnp.float32),
            jax.ShapeDtypeStruct((_N, nsub, 128), jnp.float32),
        ],
        compiler_params=pltpu.CompilerParams(
            vmem_limit_bytes=120 * 1024 * 1024),
    )(src_c, *operands)


def _edge_agg_pass(src, dst, h3, ee, denom, nheads, nsub):
    quarter = _E // 4
    alphas, parts = [], []
    for c in range(4):
        lo = c * quarter
        alpha_c, part = _edge_agg_chunk(
            jax.lax.slice(src, (lo,), (lo + quarter,)),
            jax.lax.slice(dst, (lo,), (lo + quarter,)),
            h3,
            jax.lax.slice(ee, (lo, 0, 0), (lo + quarter, 1, nheads)),
            denom, nheads, nsub, quarter)
        alphas.append(alpha_c)
        parts.append(part)
    return jnp.concatenate(alphas, axis=0), parts


def kernel(x, edge_index, W1, a_src1, a_dst1, b1, Wl1, bl1,
           W2, a_src2, a_dst2, b2, Wl2, bl2):
    src = edge_index[0]
    dst = edge_index[1]

    asrc_flat = a_src1.reshape(1, _F1)
    adst_flat = a_dst1.reshape(1, _F1)
    bfold1 = (b1 + bl1).reshape(1, _F1)
    bfold2 = (b2 + bl2).reshape(1, _C2)

    nb_grid = _N // _NB
    h1, as1, ad1, xl1 = pl.pallas_call(
        _dense1_body,
        grid=(nb_grid,),
        in_specs=[
            pl.BlockSpec((_NB, _D), lambda i: (i, 0)),
            pl.BlockSpec((_D, _F1), lambda i: (0, 0)),
            pl.BlockSpec((_D, _F1), lambda i: (0, 0)),
            pl.BlockSpec((1, _F1), lambda i: (0, 0)),
            pl.BlockSpec((1, _F1), lambda i: (0, 0)),
            pl.BlockSpec((1, _F1), lambda i: (0, 0)),
        ],
        out_specs=[
            pl.BlockSpec((_NB, _F1), lambda i: (i, 0)),
            pl.BlockSpec((_NB, _H1), lambda i: (i, 0)),
            pl.BlockSpec((_NB, _H1), lambda i: (i, 0)),
            pl.BlockSpec((_NB, _F1), lambda i: (i, 0)),
        ],
        out_shape=[
            jax.ShapeDtypeStruct((_N, _F1), jnp.float32),
            jax.ShapeDtypeStruct((_N, _H1), jnp.float32),
            jax.ShapeDtypeStruct((_N, _H1), jnp.float32),
            jax.ShapeDtypeStruct((_N, _F1), jnp.float32),
        ],
    )(x, W1, Wl1, asrc_flat, adst_flat, bfold1)

    ee1, denom1 = _edge_weight_pass(
        src, dst, as1.reshape(_N, 1, _H1), ad1.reshape(_N, 1, _H1), _H1)
    alpha1_3, g1parts = _edge_agg_pass(
        src, dst, h1.reshape(_N, 8, 128), ee1, denom1, _H1, 8)
    alpha1 = alpha1_3.reshape(_E, _H1)
    g1parts = [g.reshape(_N, _F1) for g in g1parts]

    h2, as2, ad2, xl2 = pl.pallas_call(
        _dense2_body,
        grid=(nb_grid,),
        in_specs=[
            pl.BlockSpec((_NB, _F1), lambda i: (i, 0)),
            pl.BlockSpec((_NB, _F1), lambda i: (i, 0)),
            pl.BlockSpec((_NB, _F1), lambda i: (i, 0)),
            pl.BlockSpec((_NB, _F1), lambda i: (i, 0)),
            pl.BlockSpec((_NB, _F1), lambda i: (i, 0)),
            pl.BlockSpec((_F1, _C2), lambda i: (0, 0)),
            pl.BlockSpec((_F1, _C2), lambda i: (0, 0)),
            pl.BlockSpec((1, _C2), lambda i: (0, 0)),
            pl.BlockSpec((1, _C2), lambda i: (0, 0)),
            pl.BlockSpec((1, _C2), lambda i: (0, 0)),
        ],
        out_specs=[
            pl.BlockSpec((_NB, _C2), lambda i: (i, 0)),
            pl.BlockSpec((_NB, 1), lambda i: (i, 0)),
            pl.BlockSpec((_NB, 1), lambda i: (i, 0)),
            pl.BlockSpec((_NB, _C2), lambda i: (i, 0)),
        ],
        out_shape=[
            jax.ShapeDtypeStruct((_N, _C2), jnp.float32),
            jax.ShapeDtypeStruct((_N, 1), jnp.float32),
            jax.ShapeDtypeStruct((_N, 1), jnp.float32),
            jax.ShapeDtypeStruct((_N, _C2), jnp.float32),
        ],
    )(*g1parts, xl1, W2, Wl2, a_src2, a_dst2, bfold2)

    ee2, denom2 = _edge_weight_pass(
        src, dst, as2.reshape(_N, 1, 1), ad2.reshape(_N, 1, 1), 1)
    _, g2parts = _edge_agg_pass(
        src, dst, h2.reshape(_N, 2, 128), ee2, denom2, 1, 2)
    g2parts = [g.reshape(_N, _C2) for g in g2parts]

    x2 = pl.pallas_call(
        _final_body,
        grid=(nb_grid,),
        in_specs=[pl.BlockSpec((_NB, _C2), lambda i: (i, 0))
                  for _ in range(5)],
        out_specs=pl.BlockSpec((_NB, _C2), lambda i: (i, 0)),
        out_shape=jax.ShapeDtypeStruct((_N, _C2), jnp.float32),
    )(*g2parts, xl2)

    return (x2, alpha1)
